# baseline (speedup 1.0000x reference)
import functools

import jax
import jax.numpy as jnp
from jax import lax
from jax.experimental import pallas as pl
from jax.experimental.pallas import tpu as pltpu
from jax.experimental.pallas import tpu_sc as plsc

VOCAB = 1_000_000
D = 32
B = 16384 * 50
NC, NS = 2, 16
NW = NC * NS
B_PER_W = B // NW
CHUNK = 640
NB = 4
N_CHUNKS = B_PER_W // CHUNK


def _body(idx_hbm, table_hbm, out_hbm, idx_v, rows0, rows1, rows2, rows3,
          g0, g1, g2, g3, o0, o1, o2, o3):
    wid = lax.axis_index("s") * NC + lax.axis_index("c")
    base = wid * B_PER_W
    pltpu.sync_copy(idx_hbm.at[pl.ds(base, B_PER_W)], idx_v)

    rows = (rows0, rows1, rows2, rows3)
    gsem = (g0, g1, g2, g3)
    osem = (o0, o1, o2, o3)

    def gather(c, b):
        return pltpu.make_async_copy(
            table_hbm.at[idx_v.at[pl.ds(c * CHUNK, CHUNK)]], rows[b], gsem[b])

    def out(c, b):
        return pltpu.make_async_copy(
            rows[b], out_hbm.at[pl.ds(base + c * CHUNK, CHUNK)], osem[b])

    gather(0, 0).start()
    gather(1, 1).start()
    for cc in (0, 1):
        gather(cc, cc).wait()
        out(cc, cc).start()
        gather(cc + 2, cc + 2).start()

    @pl.loop(2, N_CHUNKS - 2, step=NB)
    def _(c):
        for b in range(NB):
            cc = c + b
            bb = (2 + b) % NB
            gather(cc, bb).wait()
            out(cc, bb).start()
            nb = (bb + 2) % NB
            out(cc - 2, nb).wait()
            gather(cc + 2, nb).start()

    for cc in (N_CHUNKS - 2, N_CHUNKS - 1):
        gather(cc, cc % NB).wait()
        out(cc, cc % NB).start()
    for cc in range(N_CHUNKS - 4, N_CHUNKS):
        out(cc, cc % NB).wait()


@functools.partial(jax.jit, static_argnames=())
def _embed(idx_flat, table):
    mesh = plsc.VectorSubcoreMesh(
        core_axis_name="c", subcore_axis_name="s", num_cores=NC, num_subcores=NS)
    k = pl.kernel(
        _body,
        out_type=jax.ShapeDtypeStruct((B, D), jnp.float32),
        mesh=mesh,
        scratch_types=(
            [pltpu.VMEM((B_PER_W,), jnp.int32)]
            + [pltpu.VMEM((CHUNK, D), jnp.float32)] * NB
            + [pltpu.SemaphoreType.DMA] * (2 * NB)
        ),
        compiler_params=pltpu.CompilerParams(use_tc_tiling_on_sc=False),
    )
    return k(idx_flat, table)


def kernel(x, vocab):
    idx_flat = x.reshape(-1).astype(jnp.int32)
    out = _embed(idx_flat, vocab)
    return out.reshape(x.shape + (D,))


# speedup vs baseline: 1.6150x; 1.6150x over previous
import jax
import jax.numpy as jnp
from jax import lax
from jax.experimental import pallas as pl
from jax.experimental.pallas import tpu as pltpu
from jax.experimental.pallas import tpu_sc as plsc

VOCAB = 1_000_000
D = 32
BATCH = 16384
HIST = 50
NC, NS = 2, 16
NW = NC * NS
ROWS_W = BATCH // NW
BLK = 8
N_BLK = ROWS_W // BLK
NB = 2


def _body(x_hbm, table_hbm, out_hbm, idx_v, buf0, buf1, g0, g1, o0, o1):
    wid = lax.axis_index("s") * NC + lax.axis_index("c")
    xr0 = wid * ROWS_W
    pltpu.sync_copy(x_hbm.at[pl.ds(xr0, ROWS_W)], idx_v)

    bufs = (buf0, buf1)
    gsem = (g0, g1)
    osem = (o0, o1)

    def gathers(j, b):
        for s in range(BLK):
            yield pltpu.make_async_copy(
                table_hbm.at[idx_v.at[j * BLK + s]], bufs[b].at[s], gsem[b])

    def fire(j, b):
        for g in gathers(j, b):
            g.start()

    def drain(j, b):
        for g in gathers(j, b):
            g.wait()

    def out(j, b):
        return pltpu.make_async_copy(
            bufs[b], out_hbm.at[pl.ds(xr0 + j * BLK, BLK)], osem[b])

    fire(0, 0)
    fire(1, 1)
    drain(0, 0)
    out(0, 0).start()
    out(0, 0).wait()
    fire(2, 0)
    drain(1, 1)
    out(1, 1).start()

    @pl.loop(2, N_BLK - 2, step=2)
    def _(j):
        for b in (0, 1):
            jj = j + b
            out(jj - 1, 1 - b).wait()
            fire(jj + 1, 1 - b)
            drain(jj, b)
            out(jj, b).start()

    out(N_BLK - 3, 1).wait()
    fire(N_BLK - 1, 1)
    drain(N_BLK - 2, 0)
    out(N_BLK - 2, 0).start()
    drain(N_BLK - 1, 1)
    out(N_BLK - 1, 1).start()
    out(N_BLK - 2, 0).wait()
    out(N_BLK - 1, 1).wait()


@jax.jit
def _embed(x, table):
    mesh = plsc.VectorSubcoreMesh(
        core_axis_name="c", subcore_axis_name="s", num_cores=NC, num_subcores=NS)
    k = pl.kernel(
        _body,
        out_type=jax.ShapeDtypeStruct((BATCH, HIST, D), jnp.float32),
        mesh=mesh,
        scratch_types=(
            [pltpu.VMEM((ROWS_W, HIST), jnp.int32)]
            + [pltpu.VMEM((BLK, HIST, D), jnp.float32)] * NB
            + [pltpu.SemaphoreType.DMA] * (2 * NB)
        ),
        compiler_params=pltpu.CompilerParams(use_tc_tiling_on_sc=False),
    )
    return k(x, table)


def kernel(x, vocab):
    return _embed(x.astype(jnp.int32), vocab)


# speedup vs baseline: 3.7591x; 2.3276x over previous
import jax
import jax.numpy as jnp
from jax import lax
from jax.experimental import pallas as pl
from jax.experimental.pallas import tpu as pltpu
from jax.experimental.pallas import tpu_sc as plsc

VOCAB = 1_000_000
D = 32
BATCH = 16384
HIST = 50
NC, NS = 2, 16
D_PER_C = D // NC
B_PER_S = BATCH // NS
JH = 5
NG = HIST // JH
GH = JH * B_PER_S


def _body(xT_hbm, vT_hbm, out_hbm, idx1, vp0, plane0, psem, gsem, o0):
    cid = lax.axis_index("c")
    sid = lax.axis_index("s")
    b0 = sid * B_PER_S

    @pl.loop(0, HIST)
    def _(j):
        pltpu.sync_copy(xT_hbm.at[j, pl.ds(b0, B_PER_S)],
                        idx1.at[pl.ds(j * B_PER_S, B_PER_S)])

    def stage(d):
        return pltpu.make_async_copy(
            vT_hbm.at[cid * D_PER_C + d], plane0, psem)

    def gather(g):
        return pltpu.make_async_copy(
            plane0.at[idx1.at[pl.ds(g * GH, GH)]], vp0, gsem)

    def out_row(d, g, jl):
        dd = cid * D_PER_C + d
        return pltpu.make_async_copy(
            vp0.at[pl.ds(jl * B_PER_S, B_PER_S)],
            out_hbm.at[g * JH + jl, dd, pl.ds(b0, B_PER_S)], o0)

    for d in range(D_PER_C):
        @pl.when(sid == 0)
        def _():
            stage(d).start()
            stage(d).wait()
        plsc.subcore_barrier()

        for g in range(NG):
            if d > 0 or g > 0:
                @pl.loop(0, JH)
                def _(jl, _d=d, _g=g):
                    out_row(_d, _g, jl).wait()
            gather(g).start()
            gather(g).wait()

            @pl.loop(0, JH)
            def _(jl, _d=d, _g=g):
                out_row(_d, _g, jl).start()

        plsc.subcore_barrier()

    @pl.loop(0, JH)
    def _(jl):
        out_row(D_PER_C - 1, NG - 1, jl).wait()


@jax.jit
def _embed(xT, vT):
    mesh = plsc.VectorSubcoreMesh(
        core_axis_name="c", subcore_axis_name="s", num_cores=NC, num_subcores=NS)
    k = pl.kernel(
        _body,
        out_type=jax.ShapeDtypeStruct((HIST, D, BATCH), jnp.float32),
        mesh=mesh,
        scratch_types=[
            pltpu.VMEM((HIST * B_PER_S,), jnp.int32),
            pltpu.VMEM((GH,), jnp.float32),
            pltpu.VMEM_SHARED((VOCAB,), jnp.float32),
            pltpu.SemaphoreType.DMA,
            pltpu.SemaphoreType.DMA,
            pltpu.SemaphoreType.DMA,
        ],
        compiler_params=pltpu.CompilerParams(use_tc_tiling_on_sc=True),
    )
    return k(xT, vT)


def kernel(x, vocab):
    out_t = _embed(x.T.astype(jnp.int32), vocab.T)
    return jnp.transpose(out_t, (2, 0, 1))


# speedup vs baseline: 4.2711x; 1.1362x over previous
import jax
import jax.numpy as jnp
from jax import lax
from jax.experimental import pallas as pl
from jax.experimental.pallas import tpu as pltpu
from jax.experimental.pallas import tpu_sc as plsc

VOCAB = 1_000_000
D = 32
BATCH = 16384
HIST = 50
NC, NS = 2, 16
D_PER_C = D // NC
B_PER_S = BATCH // NS
JH = 5
NG = HIST // JH
GH = JH * B_PER_S


def _body(xT_hbm, vT_hbm, out_hbm, idx1, vp0, vp1, vp2, plane0,
          psem, g0, g1, g2, o0, o1, o2):
    cid = lax.axis_index("c")
    sid = lax.axis_index("s")
    b0 = sid * B_PER_S

    @pl.loop(0, HIST)
    def _(j):
        pltpu.sync_copy(xT_hbm.at[j, pl.ds(b0, B_PER_S)],
                        idx1.at[pl.ds(j * B_PER_S, B_PER_S)])

    def stage(d):
        return pltpu.make_async_copy(
            vT_hbm.at[cid * D_PER_C + d], plane0, psem)

    vp = (vp0, vp1, vp2)
    gsem = (g0, g1, g2)
    osem = (o0, o1, o2)

    def gather(g):
        b = g % 3
        return pltpu.make_async_copy(
            plane0.at[idx1.at[pl.ds(g * GH, GH)]], vp[b], gsem[b])

    def out_row(d, g, jl):
        dd = cid * D_PER_C + d
        b = g % 3
        return pltpu.make_async_copy(
            vp[b].at[pl.ds(jl * B_PER_S, B_PER_S)],
            out_hbm.at[g * JH + jl, dd, pl.ds(b0, B_PER_S)], osem[b])

    for d in range(D_PER_C):
        @pl.when(sid == 0)
        def _():
            stage(d).start()
            stage(d).wait()
        plsc.subcore_barrier()

        for g in range(NG + 2):
            if g < NG:
                if d > 0 or g >= 3:
                    @pl.loop(0, JH)
                    def _(jl, _d=d, _g=g):
                        out_row(_d, _g, jl).wait()
                gather(g).start()
            if g >= 2:
                gp = g - 2
                gather(gp).wait()

                @pl.loop(0, JH)
                def _(jl, _d=d, _gp=gp):
                    out_row(_d, _gp, jl).start()

        plsc.subcore_barrier()

    for g in (NG - 3, NG - 2, NG - 1):
        @pl.loop(0, JH)
        def _(jl, _g=g):
            out_row(D_PER_C - 1, _g, jl).wait()


@jax.jit
def _embed(xT, vT):
    mesh = plsc.VectorSubcoreMesh(
        core_axis_name="c", subcore_axis_name="s", num_cores=NC, num_subcores=NS)
    k = pl.kernel(
        _body,
        out_type=jax.ShapeDtypeStruct((HIST, D, BATCH), jnp.float32),
        mesh=mesh,
        scratch_types=[
            pltpu.VMEM((HIST * B_PER_S,), jnp.int32),
            pltpu.VMEM((GH,), jnp.float32),
            pltpu.VMEM((GH,), jnp.float32),
            pltpu.VMEM((GH,), jnp.float32),
            pltpu.VMEM_SHARED((VOCAB,), jnp.float32),
            pltpu.SemaphoreType.DMA,
            pltpu.SemaphoreType.DMA,
            pltpu.SemaphoreType.DMA,
            pltpu.SemaphoreType.DMA,
            pltpu.SemaphoreType.DMA,
            pltpu.SemaphoreType.DMA,
            pltpu.SemaphoreType.DMA,
        ],
        compiler_params=pltpu.CompilerParams(use_tc_tiling_on_sc=True),
    )
    return k(xT, vT)


def kernel(x, vocab):
    out_t = _embed(x.T.astype(jnp.int32), vocab.T)
    return jnp.transpose(out_t, (2, 0, 1))


# speedup vs baseline: 4.2809x; 1.0023x over previous
import jax
import jax.numpy as jnp
from jax import lax
from jax.experimental import pallas as pl
from jax.experimental.pallas import tpu as pltpu
from jax.experimental.pallas import tpu_sc as plsc

VOCAB = 1_000_000
D = 32
BATCH = 16384
HIST = 50
NC, NS = 2, 16
D_PER_C = D // NC
B_PER_S = BATCH // NS
JH = 2
NG = HIST // JH
GH = JH * B_PER_S
VP_N = 6
LAG = 4


def _body(xT_hbm, vT_hbm, out_hbm, idx1, vp0, vp1, vp2, vp3, vp4, vp5,
          plane0, psem, g0, g1, g2, g3, g4, g5, o0, o1, o2, o3, o4, o5):
    cid = lax.axis_index("c")
    sid = lax.axis_index("s")
    b0 = sid * B_PER_S

    @pl.loop(0, HIST)
    def _(j):
        pltpu.sync_copy(xT_hbm.at[j, pl.ds(b0, B_PER_S)],
                        idx1.at[pl.ds(j * B_PER_S, B_PER_S)])

    def stage(d):
        return pltpu.make_async_copy(
            vT_hbm.at[cid * D_PER_C + d], plane0, psem)

    vp = (vp0, vp1, vp2, vp3, vp4, vp5)
    gsem = (g0, g1, g2, g3, g4, g5)
    osem = (o0, o1, o2, o3, o4, o5)

    def gather(g):
        b = g % VP_N
        return pltpu.make_async_copy(
            plane0.at[idx1.at[pl.ds(g * GH, GH)]], vp[b], gsem[b])

    def out_row(d, g, jl):
        dd = cid * D_PER_C + d
        b = g % VP_N
        return pltpu.make_async_copy(
            vp[b].at[pl.ds(jl * B_PER_S, B_PER_S)],
            out_hbm.at[g * JH + jl, dd, pl.ds(b0, B_PER_S)], osem[b])

    for d in range(D_PER_C):
        @pl.when(sid == 0)
        def _():
            stage(d).start()
            stage(d).wait()
        plsc.subcore_barrier()

        for g in range(NG + LAG):
            if g < NG:
                if d > 0 or g >= VP_N:
                    @pl.loop(0, JH)
                    def _(jl, _d=d, _g=g):
                        out_row(_d, _g, jl).wait()
                gather(g).start()
            if g >= LAG:
                gp = g - LAG
                gather(gp).wait()

                @pl.loop(0, JH)
                def _(jl, _d=d, _gp=gp):
                    out_row(_d, _gp, jl).start()

        plsc.subcore_barrier()

    for g in range(NG - VP_N, NG):
        @pl.loop(0, JH)
        def _(jl, _g=g):
            out_row(D_PER_C - 1, _g, jl).wait()


@jax.jit
def _embed(xT, vT):
    mesh = plsc.VectorSubcoreMesh(
        core_axis_name="c", subcore_axis_name="s", num_cores=NC, num_subcores=NS)
    k = pl.kernel(
        _body,
        out_type=jax.ShapeDtypeStruct((HIST, D, BATCH), jnp.float32),
        mesh=mesh,
        scratch_types=[
            pltpu.VMEM((HIST * B_PER_S,), jnp.int32),
            *[pltpu.VMEM((GH,), jnp.float32) for _ in range(6)],
            pltpu.VMEM_SHARED((VOCAB,), jnp.float32),
            *[pltpu.SemaphoreType.DMA for _ in range(13)],
        ],
        compiler_params=pltpu.CompilerParams(use_tc_tiling_on_sc=True),
    )
    return k(xT, vT)


def kernel(x, vocab):
    out_t = _embed(x.T.astype(jnp.int32), vocab.T)
    return jnp.transpose(out_t, (2, 0, 1))


# speedup vs baseline: 4.6047x; 1.0756x over previous
import jax
import jax.numpy as jnp
from jax import lax
from jax.experimental import pallas as pl
from jax.experimental.pallas import tpu as pltpu
from jax.experimental.pallas import tpu_sc as plsc

VOCAB = 1_000_000
D = 32
BATCH = 16384
HIST = 50
NC, NS = 2, 16
D_PER_C = D // NC
B_PER_S = BATCH // NS
JH = 2
NG = HIST // JH
GH = JH * B_PER_S
VP_N = 6
LAG = 4


def _body(xT_hbm, vT_hbm, out_hbm, idx1, vp0, vp1, vp2, vp3, vp4, vp5,
          plane0, psem, g0, g1, g2, g3, g4, g5, o0, o1, o2, o3, o4, o5):
    cid = lax.axis_index("c")
    sid = lax.axis_index("s")
    b0 = sid * B_PER_S

    def idx_load(j):
        return pltpu.make_async_copy(
            xT_hbm.at[j, pl.ds(b0, B_PER_S)],
            idx1.at[pl.ds(j * B_PER_S, B_PER_S)], psem)

    @pl.loop(0, HIST)
    def _(j):
        idx_load(j).start()

    @pl.loop(0, HIST)
    def _(j):
        idx_load(j).wait()

    def stage(d):
        return pltpu.make_async_copy(
            vT_hbm.at[cid * D_PER_C + d], plane0, psem)

    vp = (vp0, vp1, vp2, vp3, vp4, vp5)
    gsem = (g0, g1, g2, g3, g4, g5)
    osem = (o0, o1, o2, o3, o4, o5)

    def gather(g):
        b = g % VP_N
        return pltpu.make_async_copy(
            plane0.at[idx1.at[pl.ds(g * GH, GH)]], vp[b], gsem[b])

    def out_row(d, g, jl):
        dd = cid * D_PER_C + d
        b = g % VP_N
        return pltpu.make_async_copy(
            vp[b].at[pl.ds(jl * B_PER_S, B_PER_S)],
            out_hbm.at[g * JH + jl, dd, pl.ds(b0, B_PER_S)], osem[b])

    for d in range(D_PER_C):
        @pl.when(sid == 0)
        def _():
            stage(d).start()
            stage(d).wait()
        plsc.subcore_barrier()

        for g in range(NG + LAG):
            if g < NG:
                if d > 0 or g >= VP_N:
                    @pl.loop(0, JH)
                    def _(jl, _d=d, _g=g):
                        out_row(_d, _g, jl).wait()
                gather(g).start()
            if g >= LAG:
                gp = g - LAG
                gather(gp).wait()

                @pl.loop(0, JH)
                def _(jl, _d=d, _gp=gp):
                    out_row(_d, _gp, jl).start()

        plsc.subcore_barrier()

    for g in range(NG - VP_N, NG):
        @pl.loop(0, JH)
        def _(jl, _g=g):
            out_row(D_PER_C - 1, _g, jl).wait()


@jax.jit
def _embed(xT, vT):
    mesh = plsc.VectorSubcoreMesh(
        core_axis_name="c", subcore_axis_name="s", num_cores=NC, num_subcores=NS)
    k = pl.kernel(
        _body,
        out_type=jax.ShapeDtypeStruct((HIST, D, BATCH), jnp.float32),
        mesh=mesh,
        scratch_types=[
            pltpu.VMEM((HIST * B_PER_S,), jnp.int32),
            *[pltpu.VMEM((GH,), jnp.float32) for _ in range(6)],
            pltpu.VMEM_SHARED((VOCAB,), jnp.float32),
            *[pltpu.SemaphoreType.DMA for _ in range(13)],
        ],
        compiler_params=pltpu.CompilerParams(use_tc_tiling_on_sc=True),
    )
    return k(xT, vT)


def kernel(x, vocab):
    out_t = _embed(x.T.astype(jnp.int32), vocab.T)
    return jnp.transpose(out_t, (2, 0, 1))
